# baseline (speedup 1.0000x reference)
import jax
import jax.numpy as jnp
from jax import lax
from jax.experimental import pallas as pl
from jax.experimental.pallas import tpu as pltpu
from jax.experimental.pallas import tpu_sc as plsc

M = 50000
NUM_NODE = 150
FEAT = 4
ROW_W = NUM_NODE * FEAT
N_OBS = 10000
R = 80
NCHUNK = M // R
NW = 32
TPW = (NCHUNK + NW - 1) // NW
NG = R // 16


def _body(lanes_hbm, obs_hbm, ids_hbm, proj_hbm, idx_hbm, robs_hbm,
          obs_v, ids_v, rows_v, po_v, io_v, ro_v):
    cid = lax.axis_index("c")
    sid = lax.axis_index("s")
    wid = sid * 2 + cid

    pltpu.sync_copy(obs_hbm, obs_v)

    def chunk_body(t, _):
        chunk = wid + t * NW

        @pl.when(chunk < NCHUNK)
        def _():
            r0 = chunk * R
            pltpu.sync_copy(lanes_hbm.at[pl.ds(r0 * ROW_W, R * ROW_W)], rows_v)
            pltpu.sync_copy(ids_hbm.at[pl.ds(r0, R)], ids_v)

            for g in range(NG):
                rows16 = lax.iota(jnp.int32, 16) + (g * 16)
                base = rows16 * ROW_W
                ids16 = ids_v[pl.ds(g * 16, 16)]
                oix = ids16 * 2
                ox = plsc.load_gather(obs_v, [oix])
                oy = plsc.load_gather(obs_v, [oix + 1])

                big = jnp.full((16,), 3.4e38, jnp.float32)
                i0 = jnp.ones((16,), jnp.int32)

                def node_step(k, carry):
                    best, besti = carry
                    j0 = 1 + 4 * k
                    ds_ = []
                    js_ = []
                    for u in range(4):
                        ju = j0 + u
                        cx = base + ju * 4
                        x = plsc.load_gather(rows_v, [cx])
                        y = plsc.load_gather(rows_v, [cx + 1])
                        dx = x - ox
                        dy = y - oy
                        ds_.append(dx * dx + dy * dy)
                        js_.append(jnp.full((16,), ju, jnp.int32))
                    p01 = ds_[0] <= ds_[1]
                    dA = jnp.where(p01, ds_[0], ds_[1])
                    iA = jnp.where(p01, js_[0], js_[1])
                    p23 = ds_[2] <= ds_[3]
                    dB = jnp.where(p23, ds_[2], ds_[3])
                    iB = jnp.where(p23, js_[2], js_[3])
                    pAB = dA <= dB
                    dC = jnp.where(pAB, dA, dB)
                    iC = jnp.where(pAB, iA, iB)
                    pc = dC < best
                    return (jnp.where(pc, dC, best), jnp.where(pc, iC, besti))

                _, besti = lax.fori_loop(0, 37, node_step, (big, i0))

                cm = base + besti * 4
                pxv = plsc.load_gather(rows_v, [cm - 4])
                pyv = plsc.load_gather(rows_v, [cm - 3])
                pf2 = plsc.load_gather(rows_v, [cm - 2])
                pf3 = plsc.load_gather(rows_v, [cm - 1])
                cxv = plsc.load_gather(rows_v, [cm])
                cyv = plsc.load_gather(rows_v, [cm + 1])
                cf2 = plsc.load_gather(rows_v, [cm + 2])
                cf3 = plsc.load_gather(rows_v, [cm + 3])
                nxv = plsc.load_gather(rows_v, [cm + 4])
                nyv = plsc.load_gather(rows_v, [cm + 5])
                nf2 = plsc.load_gather(rows_v, [cm + 6])
                nf3 = plsc.load_gather(rows_v, [cm + 7])

                d0 = pxv - cxv
                d1 = pyv - cyv
                d2 = pf2 - cf2
                d3 = pf3 - cf3
                dp = d0 * d0 + d1 * d1 + d2 * d2 + d3 * d3
                e0 = nxv - cxv
                e1 = nyv - cyv
                e2 = nf2 - cf2
                e3 = nf3 - cf3
                dn = e0 * e0 + e1 * e1 + e2 * e2 + e3 * e3

                p2 = dn < dp
                ib = jnp.where(p2, besti, besti - 1)
                ia = jnp.where(p2, besti + 1, besti)
                sx = jnp.where(p2, cxv, pxv)
                sy = jnp.where(p2, cyv, pyv)
                ex = jnp.where(p2, nxv, cxv)
                ey = jnp.where(p2, nyv, cyv)

                lvx = ex - sx
                lvy = ey - sy
                den = lvx * lvx + lvy * lvy
                tnum = (ox - sx) * lvx + (oy - sy) * lvy
                tt = tnum / den
                projx = sx + tt * lvx
                projy = sy + tt * lvy

                r2 = rows16 * 2
                plsc.store_scatter(po_v, [r2], projx)
                plsc.store_scatter(po_v, [r2 + 1], projy)
                plsc.store_scatter(io_v, [r2], ib)
                plsc.store_scatter(io_v, [r2 + 1], ia)
                plsc.store_scatter(ro_v, [r2], ox)
                plsc.store_scatter(ro_v, [r2 + 1], oy)

            pltpu.sync_copy(po_v, proj_hbm.at[pl.ds(r0 * 2, R * 2)])
            pltpu.sync_copy(io_v, idx_hbm.at[pl.ds(r0 * 2, R * 2)])
            pltpu.sync_copy(ro_v, robs_hbm.at[pl.ds(r0 * 2, R * 2)])

        return _

    lax.fori_loop(0, TPW, chunk_body, None)


@jax.jit
def _run(lanes, obs, ids):
    mesh = plsc.VectorSubcoreMesh(core_axis_name="c", subcore_axis_name="s")
    f = pl.kernel(
        _body,
        out_type=[
            jax.ShapeDtypeStruct((M * 2,), jnp.float32),
            jax.ShapeDtypeStruct((M * 2,), jnp.int32),
            jax.ShapeDtypeStruct((M * 2,), jnp.float32),
        ],
        mesh=mesh,
        compiler_params=pltpu.CompilerParams(
            needs_layout_passes=False, use_tc_tiling_on_sc=False),
        scratch_types=[
            pltpu.VMEM((N_OBS * 2,), jnp.float32),
            pltpu.VMEM((R,), jnp.int32),
            pltpu.VMEM((R * ROW_W,), jnp.float32),
            pltpu.VMEM((R * 2,), jnp.float32),
            pltpu.VMEM((R * 2,), jnp.int32),
            pltpu.VMEM((R * 2,), jnp.float32),
        ],
    )
    return f(lanes, obs, ids)


def kernel(lane_features, obs_pos, same_obs_mask):
    lanes = lane_features.astype(jnp.float32).reshape(M * ROW_W)
    obs = obs_pos.astype(jnp.float32).reshape(N_OBS * 2)
    ids = same_obs_mask.reshape(M)
    proj, idx, robs = _run(lanes, obs, ids)
    return proj.reshape(M, 2), idx.reshape(M, 2), robs.reshape(M, 2)


# speedup vs baseline: 51.7971x; 51.7971x over previous
import jax
import jax.numpy as jnp
from jax import lax
from jax.experimental import pallas as pl
from jax.experimental.pallas import tpu as pltpu
from jax.experimental.pallas import tpu_sc as plsc

M = 50000
M_PAD = 50048
NUM_NODE = 150
N_OBS = 10000
MB = 128
NCHUNK = M_PAD // MB
NFULL = M // MB
NW = 32
TPW = (NCHUNK + NW - 1) // NW
TILE_W = 4 * MB


def _body(lanes_hbm, obs_hbm, ids_hbm, proj_hbm, idx_hbm, robs_hbm,
          obs_v, ids_v, rows_v, po_v, io_v, ro_v):
    cid = lax.axis_index("c")
    sid = lax.axis_index("s")
    wid = sid * 2 + cid

    pltpu.sync_copy(obs_hbm, obs_v)

    def chunk_body(t, _):
        chunk = wid + t * NW

        @pl.when(chunk < NCHUNK)
        def _():
            pltpu.sync_copy(lanes_hbm.at[:, chunk, :, :], rows_v)
            pltpu.sync_copy(ids_hbm.at[pl.ds(chunk * MB, MB)], ids_v)

            def group(g):
                ml0 = g * 16
                mlv = lax.iota(jnp.int32, 16) + ml0
                ids16 = ids_v[pl.ds(ml0, 16)]
                oix = ids16 * 2
                ox = plsc.load_gather(obs_v, [oix])
                oy = plsc.load_gather(obs_v, [oix + 1])

                big = jnp.full((16,), 3.4e38, jnp.float32)
                i0 = jnp.ones((16,), jnp.int32)

                def node_step(k, carry):
                    best, besti = carry
                    j0 = 1 + 4 * k
                    ds_ = []
                    js_ = []
                    for u in range(4):
                        ju = j0 + u
                        x = rows_v[ju, 0, pl.ds(ml0, 16)]
                        y = rows_v[ju, 1, pl.ds(ml0, 16)]
                        dx = x - ox
                        dy = y - oy
                        ds_.append(dx * dx + dy * dy)
                        js_.append(jnp.full((16,), ju, jnp.int32))
                    p01 = ds_[0] <= ds_[1]
                    dA = jnp.where(p01, ds_[0], ds_[1])
                    iA = jnp.where(p01, js_[0], js_[1])
                    p23 = ds_[2] <= ds_[3]
                    dB = jnp.where(p23, ds_[2], ds_[3])
                    iB = jnp.where(p23, js_[2], js_[3])
                    pAB = dA <= dB
                    dC = jnp.where(pAB, dA, dB)
                    iC = jnp.where(pAB, iA, iB)
                    pc = dC < best
                    return (jnp.where(pc, dC, best), jnp.where(pc, iC, besti))

                _, besti = lax.fori_loop(0, 37, node_step, (big, i0))

                f0 = jnp.zeros((16,), jnp.int32)
                f1 = jnp.ones((16,), jnp.int32)
                f2i = jnp.full((16,), 2, jnp.int32)
                f3i = jnp.full((16,), 3, jnp.int32)
                pxv = plsc.load_gather(rows_v, [besti - 1, f0, mlv])
                pyv = plsc.load_gather(rows_v, [besti - 1, f1, mlv])
                pf2 = plsc.load_gather(rows_v, [besti - 1, f2i, mlv])
                pf3 = plsc.load_gather(rows_v, [besti - 1, f3i, mlv])
                cxv = plsc.load_gather(rows_v, [besti, f0, mlv])
                cyv = plsc.load_gather(rows_v, [besti, f1, mlv])
                cf2 = plsc.load_gather(rows_v, [besti, f2i, mlv])
                cf3 = plsc.load_gather(rows_v, [besti, f3i, mlv])
                nxv = plsc.load_gather(rows_v, [besti + 1, f0, mlv])
                nyv = plsc.load_gather(rows_v, [besti + 1, f1, mlv])
                nf2 = plsc.load_gather(rows_v, [besti + 1, f2i, mlv])
                nf3 = plsc.load_gather(rows_v, [besti + 1, f3i, mlv])

                d0 = pxv - cxv
                d1 = pyv - cyv
                d2 = pf2 - cf2
                d3 = pf3 - cf3
                dp = d0 * d0 + d1 * d1 + d2 * d2 + d3 * d3
                e0 = nxv - cxv
                e1 = nyv - cyv
                e2 = nf2 - cf2
                e3 = nf3 - cf3
                dn = e0 * e0 + e1 * e1 + e2 * e2 + e3 * e3

                p2 = dn < dp
                ib = jnp.where(p2, besti, besti - 1)
                ia = jnp.where(p2, besti + 1, besti)
                sx = jnp.where(p2, cxv, pxv)
                sy = jnp.where(p2, cyv, pyv)
                ex = jnp.where(p2, nxv, cxv)
                ey = jnp.where(p2, nyv, cyv)

                lvx = ex - sx
                lvy = ey - sy
                den = lvx * lvx + lvy * lvy
                tnum = (ox - sx) * lvx + (oy - sy) * lvy
                tt = tnum / den
                projx = sx + tt * lvx
                projy = sy + tt * lvy

                po_v[0, pl.ds(ml0, 16)] = projx
                po_v[1, pl.ds(ml0, 16)] = projy
                io_v[0, pl.ds(ml0, 16)] = ib
                io_v[1, pl.ds(ml0, 16)] = ia
                ro_v[0, pl.ds(ml0, 16)] = ox
                ro_v[1, pl.ds(ml0, 16)] = oy

            for g in range(5):
                group(g)
            for g in range(5, 8):
                @pl.when(chunk < NFULL)
                def _(g=g):
                    group(g)

            m0 = chunk * MB

            @pl.when(chunk < NFULL)
            def _():
                pltpu.sync_copy(po_v, proj_hbm.at[:, pl.ds(m0, MB)])
                pltpu.sync_copy(io_v, idx_hbm.at[:, pl.ds(m0, MB)])
                pltpu.sync_copy(ro_v, robs_hbm.at[:, pl.ds(m0, MB)])

            @pl.when(chunk == NFULL)
            def _():
                pltpu.sync_copy(po_v.at[:, pl.ds(0, 80)],
                                proj_hbm.at[:, pl.ds(NFULL * MB, 80)])
                pltpu.sync_copy(io_v.at[:, pl.ds(0, 80)],
                                idx_hbm.at[:, pl.ds(NFULL * MB, 80)])
                pltpu.sync_copy(ro_v.at[:, pl.ds(0, 80)],
                                robs_hbm.at[:, pl.ds(NFULL * MB, 80)])

        return _

    lax.fori_loop(0, TPW, chunk_body, None)


@jax.jit
def _run(lanes, obs, ids):
    mesh = plsc.VectorSubcoreMesh(core_axis_name="c", subcore_axis_name="s")
    f = pl.kernel(
        _body,
        out_type=[
            jax.ShapeDtypeStruct((2, M), jnp.float32),
            jax.ShapeDtypeStruct((2, M), jnp.int32),
            jax.ShapeDtypeStruct((2, M), jnp.float32),
        ],
        mesh=mesh,
        compiler_params=pltpu.CompilerParams(
            needs_layout_passes=False, use_tc_tiling_on_sc=False),
        scratch_types=[
            pltpu.VMEM((N_OBS * 2,), jnp.float32),
            pltpu.VMEM((MB,), jnp.int32),
            pltpu.VMEM((NUM_NODE, 4, MB), jnp.float32),
            pltpu.VMEM((2, MB), jnp.float32),
            pltpu.VMEM((2, MB), jnp.int32),
            pltpu.VMEM((2, MB), jnp.float32),
        ],
    )
    return f(lanes, obs, ids)


def kernel(lane_features, obs_pos, same_obs_mask):
    ids = same_obs_mask.reshape(M)
    c = jnp.where(ids[0] < 2 ** 30, jnp.float32(1.0), jnp.float32(2.0))
    lt = lane_features.astype(jnp.float32).transpose(1, 2, 0)
    ltp = jnp.pad(lt, ((0, 0), (0, 0), (0, M_PAD - M)))
    lanes = (ltp.reshape(NUM_NODE, 4, NCHUNK, MB)
             .transpose(0, 2, 1, 3)) * c
    obs = obs_pos.astype(jnp.float32).reshape(N_OBS * 2)
    ids_p = jnp.pad(ids, (0, M_PAD - M))
    proj, idx, robs = _run(lanes, obs, ids_p)
    return proj.T, idx.T, robs.T
